# baseline (speedup 1.0000x reference)
import functools

import jax
import jax.numpy as jnp
from jax import lax
from jax.experimental import pallas as pl
from jax.experimental.pallas import tpu as pltpu


def _conv_bn_relu_kernel(x_ref, w_ref, b_ref, m_ref, o_ref,
                         xc_ref, xl_ref, xr_ref, *, H, W, Cin, Cout):
    M = H * W

    @pl.when(pl.program_id(0) == 0)
    def _():
        xc_ref[:W] = jnp.zeros((W, Cin), jnp.bfloat16)
        xc_ref[W + M:] = jnp.zeros((W, Cin), jnp.bfloat16)
        xl_ref[:W] = jnp.zeros((W, Cin), jnp.bfloat16)
        xl_ref[W + M:] = jnp.zeros((W, Cin), jnp.bfloat16)
        xr_ref[:W] = jnp.zeros((W, Cin), jnp.bfloat16)
        xr_ref[W + M:] = jnp.zeros((W, Cin), jnp.bfloat16)

    xc_ref[pl.ds(W, M)] = x_ref[0].reshape(M, Cin).astype(jnp.bfloat16)

    mask_l = m_ref[:, 0:1] != 0
    mask_r = m_ref[:, 1:2] != 0
    xl_ref[pl.ds(W, M)] = jnp.where(mask_l, xc_ref[pl.ds(W - 1, M)],
                                    jnp.bfloat16(0))
    xr_ref[pl.ds(W, M)] = jnp.where(mask_r, xc_ref[pl.ds(W + 1, M)],
                                    jnp.bfloat16(0))

    acc = jnp.zeros((M, Cout), jnp.float32)
    for kw, buf in enumerate((xl_ref, xc_ref, xr_ref)):
        for kh in range(3):
            acc = acc + jnp.dot(buf[pl.ds(kh * W, M)], w_ref[kh * 3 + kw],
                                preferred_element_type=jnp.float32)

    y = jnp.maximum(acc + b_ref[...], 0.0)
    o_ref[0] = y.reshape(H, W, Cout).astype(o_ref.dtype)


def kernel(x, weight, gamma, beta, running_mean, running_var):
    B, Cin, H, W = x.shape
    Cout, Cin_w, KH, KW = weight.shape
    assert (Cin_w, KH, KW) == (Cin, 3, 3)
    M = H * W

    inv = gamma / jnp.sqrt(running_var + 1e-5)
    shift = (beta - running_mean * inv).astype(jnp.float32)
    w_scaled = weight * inv[:, None, None, None]
    w_taps = jnp.transpose(w_scaled, (2, 3, 1, 0)).reshape(
        KH * KW, Cin, Cout).astype(jnp.bfloat16)

    wo = jnp.arange(M, dtype=jnp.int32) % W
    masks = jnp.stack([(wo >= 1), (wo <= W - 2)], axis=1).astype(jnp.int32)

    x_nhwc = jnp.transpose(x, (0, 2, 3, 1))

    out = pl.pallas_call(
        functools.partial(_conv_bn_relu_kernel, H=H, W=W, Cin=Cin, Cout=Cout),
        out_shape=jax.ShapeDtypeStruct((B, H, W, Cout), x.dtype),
        grid=(B,),
        in_specs=[
            pl.BlockSpec((1, H, W, Cin), lambda b: (b, 0, 0, 0)),
            pl.BlockSpec((KH * KW, Cin, Cout), lambda b: (0, 0, 0)),
            pl.BlockSpec((1, Cout), lambda b: (0, 0)),
            pl.BlockSpec((M, 2), lambda b: (0, 0)),
        ],
        out_specs=pl.BlockSpec((1, H, W, Cout), lambda b: (b, 0, 0, 0)),
        scratch_shapes=[
            pltpu.VMEM((M + 2 * W, Cin), jnp.bfloat16),
            pltpu.VMEM((M + 2 * W, Cin), jnp.bfloat16),
            pltpu.VMEM((M + 2 * W, Cin), jnp.bfloat16),
        ],
        compiler_params=pltpu.CompilerParams(
            dimension_semantics=("parallel",)),
    )(x_nhwc, w_taps, shift.reshape(1, Cout), masks)

    return jnp.transpose(out, (0, 3, 1, 2))


# speedup vs baseline: 1.5493x; 1.5493x over previous
import functools

import jax
import jax.numpy as jnp
from jax import lax
from jax.experimental import pallas as pl
from jax.experimental.pallas import tpu as pltpu


def _conv_bn_relu_kernel(x_ref, w_ref, b_ref, m_ref, o_ref, *,
                         H, W, Cin, Cout):
    M = H * W

    mask_l = m_ref[:, 0:1] != 0
    mask_r = m_ref[:, 1:2] != 0

    for i in range(x_ref.shape[0]):
        x = x_ref[i].reshape(M, Cin).astype(jnp.bfloat16)
        acc = jnp.zeros((M, Cout), jnp.float32)
        for kh in range(3):
            for kw in range(3):
                s = (kh - 1) * W + (kw - 1)
                if s > 0:
                    xs = jnp.concatenate(
                        [x[s:], jnp.zeros((s, Cin), jnp.bfloat16)], axis=0)
                elif s < 0:
                    xs = jnp.concatenate(
                        [jnp.zeros((-s, Cin), jnp.bfloat16), x[:M + s]],
                        axis=0)
                else:
                    xs = x
                if kw == 0:
                    xs = jnp.where(mask_l, xs, jnp.bfloat16(0))
                elif kw == 2:
                    xs = jnp.where(mask_r, xs, jnp.bfloat16(0))
                acc = acc + jnp.dot(xs, w_ref[kh * 3 + kw],
                                    preferred_element_type=jnp.float32)

        y = jnp.maximum(acc + b_ref[...], 0.0)
        o_ref[i] = y.reshape(H, W, Cout).astype(o_ref.dtype)


def kernel(x, weight, gamma, beta, running_mean, running_var):
    B, Cin, H, W = x.shape
    Cout, Cin_w, KH, KW = weight.shape
    assert (Cin_w, KH, KW) == (Cin, 3, 3)
    M = H * W

    inv = gamma / jnp.sqrt(running_var + 1e-5)
    shift = (beta - running_mean * inv).astype(jnp.float32)
    w_scaled = weight * inv[:, None, None, None]
    w_taps = jnp.transpose(w_scaled, (2, 3, 1, 0)).reshape(
        KH * KW, Cin, Cout).astype(jnp.bfloat16)

    wo = jnp.arange(M, dtype=jnp.int32) % W
    masks = jnp.stack([(wo >= 1), (wo <= W - 2)], axis=1).astype(jnp.int32)

    x_nhwc = jnp.transpose(x, (0, 2, 3, 1))

    out = pl.pallas_call(
        functools.partial(_conv_bn_relu_kernel, H=H, W=W, Cin=Cin, Cout=Cout),
        out_shape=jax.ShapeDtypeStruct((B, H, W, Cout), x.dtype),
        grid=(B // 2,),
        in_specs=[
            pl.BlockSpec((2, H, W, Cin), lambda b: (b, 0, 0, 0)),
            pl.BlockSpec((KH * KW, Cin, Cout), lambda b: (0, 0, 0)),
            pl.BlockSpec((1, Cout), lambda b: (0, 0)),
            pl.BlockSpec((M, 2), lambda b: (0, 0)),
        ],
        out_specs=pl.BlockSpec((2, H, W, Cout), lambda b: (b, 0, 0, 0)),
        compiler_params=pltpu.CompilerParams(
            dimension_semantics=("parallel",)),
    )(x_nhwc, w_taps, shift.reshape(1, Cout), masks)

    return jnp.transpose(out, (0, 3, 1, 2))


# speedup vs baseline: 1.6342x; 1.0548x over previous
import functools

import jax
import jax.numpy as jnp
from jax import lax
from jax.experimental import pallas as pl
from jax.experimental.pallas import tpu as pltpu


def _conv_bn_relu_kernel(x_ref, w_ref, b_ref, m_ref, o_ref, *,
                         H, W, Cin, Cout):
    M = H * W

    mask_l = m_ref[:, 0:1] != 0
    mask_r = m_ref[:, 1:2] != 0

    for i in range(x_ref.shape[0]):
        x = x_ref[i].reshape(M, Cin).astype(jnp.bfloat16)
        acc = jnp.zeros((M, Cout), jnp.float32)
        for kh in range(3):
            for kw in range(3):
                s = (kh - 1) * W + (kw - 1)
                if s > 0:
                    xs = jnp.concatenate(
                        [x[s:], jnp.zeros((s, Cin), jnp.bfloat16)], axis=0)
                elif s < 0:
                    xs = jnp.concatenate(
                        [jnp.zeros((-s, Cin), jnp.bfloat16), x[:M + s]],
                        axis=0)
                else:
                    xs = x
                if kw == 0:
                    xs = jnp.where(mask_l, xs, jnp.bfloat16(0))
                elif kw == 2:
                    xs = jnp.where(mask_r, xs, jnp.bfloat16(0))
                acc = acc + jnp.dot(xs, w_ref[kh * 3 + kw],
                                    preferred_element_type=jnp.float32)

        y = jnp.maximum(acc + b_ref[...], 0.0)
        o_ref[i] = y.reshape(H, W, Cout).astype(o_ref.dtype)


def kernel(x, weight, gamma, beta, running_mean, running_var):
    B, Cin, H, W = x.shape
    Cout, Cin_w, KH, KW = weight.shape
    assert (Cin_w, KH, KW) == (Cin, 3, 3)
    M = H * W

    inv = gamma / jnp.sqrt(running_var + 1e-5)
    shift = (beta - running_mean * inv).astype(jnp.float32)
    w_scaled = weight * inv[:, None, None, None]
    w_taps = jnp.transpose(w_scaled, (2, 3, 1, 0)).reshape(
        KH * KW, Cin, Cout).astype(jnp.bfloat16)

    wo = jnp.arange(M, dtype=jnp.int32) % W
    masks = jnp.stack([(wo >= 1), (wo <= W - 2)], axis=1).astype(jnp.int32)

    x_nhwc = jnp.transpose(x, (0, 2, 3, 1))

    out = pl.pallas_call(
        functools.partial(_conv_bn_relu_kernel, H=H, W=W, Cin=Cin, Cout=Cout),
        out_shape=jax.ShapeDtypeStruct((B, H, W, Cout), x.dtype),
        grid=(B // 4,),
        in_specs=[
            pl.BlockSpec((4, H, W, Cin), lambda b: (b, 0, 0, 0)),
            pl.BlockSpec((KH * KW, Cin, Cout), lambda b: (0, 0, 0)),
            pl.BlockSpec((1, Cout), lambda b: (0, 0)),
            pl.BlockSpec((M, 2), lambda b: (0, 0)),
        ],
        out_specs=pl.BlockSpec((4, H, W, Cout), lambda b: (b, 0, 0, 0)),
        compiler_params=pltpu.CompilerParams(
            dimension_semantics=("parallel",)),
    )(x_nhwc, w_taps, shift.reshape(1, Cout), masks)

    return jnp.transpose(out, (0, 3, 1, 2))


# speedup vs baseline: 1.6431x; 1.0055x over previous
import functools

import jax
import jax.numpy as jnp
from jax import lax
from jax.experimental import pallas as pl
from jax.experimental.pallas import tpu as pltpu


def _conv_bn_relu_kernel(x_ref, w_ref, b_ref, m_ref, o_ref, *,
                         H, W, Cin, Cout):
    M = H * W

    mask_l = m_ref[:, 0:1] != 0
    mask_r = m_ref[:, 1:2] != 0

    for i in range(x_ref.shape[0]):
        x = x_ref[i].reshape(M, Cin)
        acc = jnp.zeros((M, Cout), jnp.float32)
        for kh in range(3):
            for kw in range(3):
                s = (kh - 1) * W + (kw - 1)
                if s > 0:
                    xs = jnp.concatenate(
                        [x[s:], jnp.zeros((s, Cin), jnp.float32)], axis=0)
                elif s < 0:
                    xs = jnp.concatenate(
                        [jnp.zeros((-s, Cin), jnp.float32), x[:M + s]],
                        axis=0)
                else:
                    xs = x
                if kw == 0:
                    xs = jnp.where(mask_l, xs, 0.0)
                elif kw == 2:
                    xs = jnp.where(mask_r, xs, 0.0)
                acc = acc + jnp.dot(xs, w_ref[kh * 3 + kw],
                                    preferred_element_type=jnp.float32)

        y = jnp.maximum(acc + b_ref[...], 0.0)
        o_ref[i] = y.reshape(H, W, Cout).astype(o_ref.dtype)


def kernel(x, weight, gamma, beta, running_mean, running_var):
    B, Cin, H, W = x.shape
    Cout, Cin_w, KH, KW = weight.shape
    assert (Cin_w, KH, KW) == (Cin, 3, 3)
    M = H * W

    inv = gamma / jnp.sqrt(running_var + 1e-5)
    shift = (beta - running_mean * inv).astype(jnp.float32)
    w_scaled = weight * inv[:, None, None, None]
    w_taps = jnp.transpose(w_scaled, (2, 3, 1, 0)).reshape(
        KH * KW, Cin, Cout)

    wo = jnp.arange(M, dtype=jnp.int32) % W
    masks = jnp.stack([(wo >= 1), (wo <= W - 2)], axis=1).astype(jnp.int32)

    x_nhwc = jnp.transpose(x, (0, 2, 3, 1))

    out = pl.pallas_call(
        functools.partial(_conv_bn_relu_kernel, H=H, W=W, Cin=Cin, Cout=Cout),
        out_shape=jax.ShapeDtypeStruct((B, H, W, Cout), x.dtype),
        grid=(B // 4,),
        in_specs=[
            pl.BlockSpec((4, H, W, Cin), lambda b: (b, 0, 0, 0)),
            pl.BlockSpec((KH * KW, Cin, Cout), lambda b: (0, 0, 0)),
            pl.BlockSpec((1, Cout), lambda b: (0, 0)),
            pl.BlockSpec((M, 2), lambda b: (0, 0)),
        ],
        out_specs=pl.BlockSpec((4, H, W, Cout), lambda b: (b, 0, 0, 0)),
        compiler_params=pltpu.CompilerParams(
            dimension_semantics=("parallel",)),
    )(x_nhwc, w_taps, shift.reshape(1, Cout), masks)

    return jnp.transpose(out, (0, 3, 1, 2))
